# baseline (speedup 1.0000x reference)
import functools
import math

import jax
import jax.numpy as jnp
from jax import lax
from jax.experimental import pallas as pl
from jax.experimental.pallas import tpu as pltpu
from jax.experimental.pallas import tpu_sc as plsc

_SIZE = 32000
_ROWS = 4096
_SMOOTH = 0.1
_CONF = 1.0 - _SMOOTH
_EPS = _SMOOTH / (_SIZE - 2)
_C_CONST = (_SIZE - 2) * _EPS * math.log(_EPS) + _CONF * math.log(_CONF)

_RB = 512
_CB = 5376
_C1 = 21504
_W = _SIZE - _C1

_NW = 32
_RPW = _ROWS // _NW
_L = 16
_KR = 8
_UNR = 8


def _sc_body(x_hbm, tsp_hbm, out_hbm, tsp_v, buf, acc_v, sem):
    wid = lax.axis_index("s") * 2 + lax.axis_index("c")
    base = wid * _RPW
    pltpu.sync_copy(tsp_hbm.at[pl.ds(base * _L, _RPW * _L)], tsp_v)
    lane = lax.iota(jnp.int32, _L)
    zero = jnp.zeros((_L,), jnp.float32)

    def blk_step(b, acc):
        pltpu.async_copy(
            x_hbm.at[pl.ds(base + b * _KR, _KR), pl.ds(_C1, _W)], buf, sem
        ).wait()
        for r in range(_KR):
            t_splat = tsp_v[pl.ds((b * _KR + r) * _L, _L)]
            t_shift = t_splat - (_C1 + lane)

            def col_step(k0, carry):
                ss = list(carry[:4])
                gg = list(carry[4:])
                for j in range(_UNR):
                    k = k0 * _UNR + j
                    x16 = buf[r, pl.ds(k * _L, _L)]
                    ss[j % 4] = ss[j % 4] + x16
                    gg[j % 2] = gg[j % 2] + jnp.where(
                        t_shift == k * _L, x16, 0.0
                    )
                return tuple(ss) + tuple(gg)

            parts = lax.fori_loop(
                0, _W // (_L * _UNR), col_step, (zero,) * 6
            )
            s = parts[0] + parts[1] + parts[2] + parts[3]
            g = parts[4] + parts[5]
            acc = acc + jnp.where(
                t_splat != 0, -_EPS * s + (_EPS - _CONF) * g, zero
            )
        return acc

    acc = lax.fori_loop(0, _RPW // _KR, blk_step, zero)
    acc_v[...] = acc
    pltpu.sync_copy(acc_v, out_hbm.at[wid])


_sc_stripe = functools.partial(
    pl.kernel,
    mesh=plsc.VectorSubcoreMesh(core_axis_name="c", subcore_axis_name="s"),
    out_type=jax.ShapeDtypeStruct((_NW, _L), jnp.float32),
    scratch_types=[
        pltpu.VMEM((_RPW * _L,), jnp.int32),
        pltpu.VMEM((_KR, _W), jnp.float32),
        pltpu.VMEM((_L,), jnp.float32),
        pltpu.SemaphoreType.DMA,
    ],
)(_sc_body)


def _reduce_body(tgt_ref, x_ref, out_ref):
    i = pl.program_id(0)
    j = pl.program_id(1)

    @pl.when((i == 0) & (j == 0))
    def _init():
        out_ref[...] = jnp.zeros_like(out_ref)

    tgt = tgt_ref[...]
    valid = (tgt != 0).astype(jnp.float32)
    xb = x_ref[...]

    w_eps = -_EPS * valid
    w_conf = -_CONF * valid
    t_shift = tgt - j * _CB
    col_ids = jax.lax.broadcasted_iota(jnp.int32, (_RB, _CB), 1)
    acc = jnp.sum(xb * jnp.where(col_ids == t_shift, w_conf, w_eps))

    col0 = jnp.sum(xb[:, 0:1] * valid)
    nvalid = jnp.sum(valid)
    acc = acc + jnp.where(j == 0, _EPS * col0 + _C_CONST * nvalid, 0.0)

    out_ref[...] += acc


def _combine_body(tc_ref, sc_ref, out_ref):
    out_ref[...] = tc_ref[...] + jnp.sum(sc_ref[...])


@jax.jit
def kernel(x, target):
    tgt = target.astype(jnp.int32)
    tsp = jnp.repeat(tgt, _L)
    sc_part = _sc_stripe(x, tsp)
    tc_part = pl.pallas_call(
        _reduce_body,
        grid=(_ROWS // _RB, _C1 // _CB),
        in_specs=[
            pl.BlockSpec((_RB, 1), lambda i, j: (i, 0)),
            pl.BlockSpec((_RB, _CB), lambda i, j: (i, j)),
        ],
        out_specs=pl.BlockSpec((1, 1), lambda i, j: (0, 0)),
        out_shape=jax.ShapeDtypeStruct((1, 1), jnp.float32),
    )(tgt.reshape(_ROWS, 1), x)
    out = pl.pallas_call(
        _combine_body,
        out_shape=jax.ShapeDtypeStruct((1, 1), jnp.float32),
    )(tc_part, sc_part)
    return out[0, 0]


# speedup vs baseline: 1.0172x; 1.0172x over previous
import functools
import math

import jax
import jax.numpy as jnp
from jax import lax
from jax.experimental import pallas as pl
from jax.experimental.pallas import tpu as pltpu
from jax.experimental.pallas import tpu_sc as plsc

_SIZE = 32000
_ROWS = 4096
_SMOOTH = 0.1
_CONF = 1.0 - _SMOOTH
_EPS = _SMOOTH / (_SIZE - 2)
_C_CONST = (_SIZE - 2) * _EPS * math.log(_EPS) + _CONF * math.log(_CONF)

_RB = 512
_CB = 5632
_C1 = 22528
_W = _SIZE - _C1

_NW = 32
_RPW = _ROWS // _NW
_L = 16
_KR = 8
_UNR = 8


def _sc_body(x_hbm, tsp_hbm, out_hbm, tsp_v, buf, acc_v, sem):
    wid = lax.axis_index("s") * 2 + lax.axis_index("c")
    base = wid * _RPW
    pltpu.sync_copy(tsp_hbm.at[pl.ds(base * _L, _RPW * _L)], tsp_v)
    lane = lax.iota(jnp.int32, _L)
    zero = jnp.zeros((_L,), jnp.float32)

    def blk_step(b, acc):
        pltpu.async_copy(
            x_hbm.at[pl.ds(base + b * _KR, _KR), pl.ds(_C1, _W)], buf, sem
        ).wait()
        for r in range(_KR):
            t_splat = tsp_v[pl.ds((b * _KR + r) * _L, _L)]
            t_shift = t_splat - (_C1 + lane)

            def col_step(k0, carry):
                ss = list(carry[:4])
                gg = list(carry[4:])
                for j in range(_UNR):
                    k = k0 * _UNR + j
                    x16 = buf[r, pl.ds(k * _L, _L)]
                    ss[j % 4] = ss[j % 4] + x16
                    gg[j % 2] = gg[j % 2] + jnp.where(
                        t_shift == k * _L, x16, 0.0
                    )
                return tuple(ss) + tuple(gg)

            parts = lax.fori_loop(
                0, _W // (_L * _UNR), col_step, (zero,) * 6
            )
            s = parts[0] + parts[1] + parts[2] + parts[3]
            g = parts[4] + parts[5]
            acc = acc + jnp.where(
                t_splat != 0, -_EPS * s + (_EPS - _CONF) * g, zero
            )
        return acc

    acc = lax.fori_loop(0, _RPW // _KR, blk_step, zero)
    acc_v[...] = acc
    pltpu.sync_copy(acc_v, out_hbm.at[wid])


_sc_stripe = functools.partial(
    pl.kernel,
    mesh=plsc.VectorSubcoreMesh(core_axis_name="c", subcore_axis_name="s"),
    out_type=jax.ShapeDtypeStruct((_NW, _L), jnp.float32),
    scratch_types=[
        pltpu.VMEM((_RPW * _L,), jnp.int32),
        pltpu.VMEM((_KR, _W), jnp.float32),
        pltpu.VMEM((_L,), jnp.float32),
        pltpu.SemaphoreType.DMA,
    ],
)(_sc_body)


def _reduce_body(tgt_ref, x_ref, out_ref):
    i = pl.program_id(0)
    j = pl.program_id(1)

    @pl.when((i == 0) & (j == 0))
    def _init():
        out_ref[...] = jnp.zeros_like(out_ref)

    tgt = tgt_ref[...]
    valid = (tgt != 0).astype(jnp.float32)
    xb = x_ref[...]

    w_eps = -_EPS * valid
    w_conf = -_CONF * valid
    t_shift = tgt - j * _CB
    col_ids = jax.lax.broadcasted_iota(jnp.int32, (_RB, _CB), 1)
    acc = jnp.sum(xb * jnp.where(col_ids == t_shift, w_conf, w_eps))

    col0 = jnp.sum(xb[:, 0:1] * valid)
    nvalid = jnp.sum(valid)
    acc = acc + jnp.where(j == 0, _EPS * col0 + _C_CONST * nvalid, 0.0)

    out_ref[...] += acc


def _combine_body(tc_ref, sc_ref, out_ref):
    out_ref[...] = tc_ref[...] + jnp.sum(sc_ref[...])


@jax.jit
def kernel(x, target):
    tgt = target.astype(jnp.int32)
    tsp = jnp.repeat(tgt, _L)
    sc_part = _sc_stripe(x, tsp)
    tc_part = pl.pallas_call(
        _reduce_body,
        grid=(_ROWS // _RB, _C1 // _CB),
        in_specs=[
            pl.BlockSpec((_RB, 1), lambda i, j: (i, 0)),
            pl.BlockSpec((_RB, _CB), lambda i, j: (i, j)),
        ],
        out_specs=pl.BlockSpec((1, 1), lambda i, j: (0, 0)),
        out_shape=jax.ShapeDtypeStruct((1, 1), jnp.float32),
    )(tgt.reshape(_ROWS, 1), x)
    out = pl.pallas_call(
        _combine_body,
        out_shape=jax.ShapeDtypeStruct((1, 1), jnp.float32),
    )(tc_part, sc_part)
    return out[0, 0]
